# speedup vs baseline: 19.4910x; 19.4910x over previous
import functools

import jax
import jax.numpy as jnp
from jax import lax
from jax.experimental import pallas as pl
from jax.experimental.pallas import tpu as pltpu
from jax.experimental.pallas import tpu_sc as plsc

NC = 2
NS = 16
NW = NC * NS
CHUNK = 128

_MESH = dict(core_axis_name="c", subcore_axis_name="s", num_cores=NC,
             num_subcores=NS)


def _deg_kernel_body(cpw, rpt, col_hbm, deg_out, colv, ones_v, zer_v, hist):
  cid = lax.axis_index("c")
  sid = lax.axis_index("s")
  wid = sid * NC + cid

  for i in range(CHUNK // 16):
    ones_v[pl.ds(i * 16, 16)] = jnp.ones((16,), jnp.float32)
  for i in range(rpt // 16):
    zer_v[pl.ds(i * 16, 16)] = jnp.zeros((16,), jnp.float32)
  pltpu.sync_copy(zer_v, hist.at[pl.ds(sid * rpt, rpt)])
  pltpu.sync_copy(col_hbm.at[wid], colv)
  plsc.subcore_barrier()

  def body(j, carry):
    pltpu.sync_copy(ones_v, hist.at[colv.at[j]], add=True)
    return carry

  lax.fori_loop(0, cpw, body, 0)
  plsc.subcore_barrier()
  pltpu.sync_copy(hist.at[pl.ds(sid * rpt, rpt)],
                  deg_out.at[cid, pl.ds(sid * rpt, rpt)])


def _agg_kernel_body(cpw, rpt, y_hbm, row_hbm, col_hbm, z_hbm, out_hbm,
                     rowv, colv, buf, acc, sem):
  cid = lax.axis_index("c")
  sid = lax.axis_index("s")
  wid = sid * NC + cid

  pltpu.sync_copy(z_hbm, acc.at[pl.ds(sid * rpt, rpt)])
  pltpu.sync_copy(row_hbm.at[wid], rowv)
  pltpu.sync_copy(col_hbm.at[wid], colv)
  plsc.subcore_barrier()

  def body(j, carry):
    pltpu.async_copy(y_hbm.at[rowv.at[j]], buf, sem).wait()
    pltpu.sync_copy(buf, acc.at[colv.at[j]], add=True)
    return carry

  lax.fori_loop(0, cpw, body, 0)
  plsc.subcore_barrier()
  pltpu.sync_copy(acc.at[pl.ds(sid * rpt, rpt)],
                  out_hbm.at[cid, pl.ds(sid * rpt, rpt)])


def _matmul_body(xb, degb, wb, yb):
  deg = degb[0, :] + degb[1, :] + 1.0
  dis = lax.rsqrt(deg)
  yb[...] = jnp.dot(xb[...], wb[...],
                    preferred_element_type=jnp.float32) * dis[:, None]


def _finish_body(accb, yb, degb, bb, ab, zb):
  deg = degb[0, :] + degb[1, :] + 1.0
  dis = lax.rsqrt(deg)
  s = (accb[0] + accb[1] + yb[...]) * dis[:, None] + bb[...]
  zb[...] = jnp.where(s >= 0, s, ab[...] * s)


def kernel(x, edge_index, W, b, alpha):
  n = x.shape[0]
  e = edge_index.shape[1]
  d = x.shape[1]

  nacc = 10240
  rpt = nacc // NS
  cpw = -(-e // (NW * CHUNK))
  epad = NW * cpw * CHUNK - e

  row = jnp.concatenate(
      [edge_index[0], jnp.zeros((epad,), jnp.int32)]).reshape(NW, cpw, CHUNK)
  col = jnp.concatenate(
      [edge_index[1], jnp.full((epad,), n, jnp.int32)]).reshape(NW, cpw, CHUNK)
  x_pad = jnp.pad(x, ((0, nacc - n), (0, 0)))
  zeros_rows = jnp.zeros((rpt, d), jnp.float32)

  mesh = plsc.VectorSubcoreMesh(**_MESH)

  deg_fn = pl.kernel(
      functools.partial(_deg_kernel_body, cpw, rpt),
      out_type=jax.ShapeDtypeStruct((NC, nacc), jnp.float32),
      mesh=mesh,
      scratch_types=[
          pltpu.VMEM((cpw, CHUNK), jnp.int32),
          pltpu.VMEM((CHUNK,), jnp.float32),
          pltpu.VMEM((rpt,), jnp.float32),
          pltpu.VMEM_SHARED((nacc,), jnp.float32),
      ])
  deg = deg_fn(col)

  y = pl.pallas_call(
      _matmul_body,
      grid=(nacc // 1024,),
      in_specs=[
          pl.BlockSpec((1024, d), lambda i: (i, 0)),
          pl.BlockSpec((NC, 1024), lambda i: (0, i)),
          pl.BlockSpec((d, d), lambda i: (0, 0)),
      ],
      out_specs=pl.BlockSpec((1024, d), lambda i: (i, 0)),
      out_shape=jax.ShapeDtypeStruct((nacc, d), jnp.float32),
  )(x_pad, deg, W)

  agg_fn = pl.kernel(
      functools.partial(_agg_kernel_body, cpw, rpt),
      out_type=jax.ShapeDtypeStruct((NC, nacc, d), jnp.float32),
      mesh=mesh,
      scratch_types=[
          pltpu.VMEM((cpw, CHUNK), jnp.int32),
          pltpu.VMEM((cpw, CHUNK), jnp.int32),
          pltpu.VMEM((CHUNK, d), jnp.float32),
          pltpu.VMEM_SHARED((nacc, d), jnp.float32),
          pltpu.SemaphoreType.DMA,
      ])
  accp = agg_fn(y, row, col, zeros_rows)

  z = pl.pallas_call(
      _finish_body,
      grid=(nacc // 1024,),
      in_specs=[
          pl.BlockSpec((NC, 1024, d), lambda i: (0, i, 0)),
          pl.BlockSpec((1024, d), lambda i: (i, 0)),
          pl.BlockSpec((NC, 1024), lambda i: (0, i)),
          pl.BlockSpec((1, d), lambda i: (0, 0)),
          pl.BlockSpec((1, d), lambda i: (0, 0)),
      ],
      out_specs=pl.BlockSpec((1024, d), lambda i: (i, 0)),
      out_shape=jax.ShapeDtypeStruct((nacc, d), jnp.float32),
  )(accp, y, deg, b.reshape(1, d), alpha.reshape(1, d))

  return z[:n]


# speedup vs baseline: 27.0802x; 1.3894x over previous
import functools

import jax
import jax.numpy as jnp
from jax import lax
from jax.experimental import pallas as pl
from jax.experimental.pallas import tpu as pltpu
from jax.experimental.pallas import tpu_sc as plsc

NC = 2
NS = 16
NW = NC * NS
CHUNK = 128

_MESH = dict(core_axis_name="c", subcore_axis_name="s", num_cores=NC,
             num_subcores=NS)


def _deg_kernel_body(cpw, rpt, col_hbm, deg_out, colv, ones_v, zer_v, hist):
  cid = lax.axis_index("c")
  sid = lax.axis_index("s")
  wid = sid * NC + cid

  for i in range(CHUNK // 16):
    ones_v[pl.ds(i * 16, 16)] = jnp.ones((16,), jnp.float32)
  for i in range(rpt // 16):
    zer_v[pl.ds(i * 16, 16)] = jnp.zeros((16,), jnp.float32)
  pltpu.sync_copy(zer_v, hist.at[pl.ds(sid * rpt, rpt)])
  pltpu.sync_copy(col_hbm.at[wid], colv)
  plsc.subcore_barrier()

  def body(j, carry):
    pltpu.sync_copy(ones_v, hist.at[colv.at[j]], add=True)
    return carry

  lax.fori_loop(0, cpw, body, 0)
  plsc.subcore_barrier()
  pltpu.sync_copy(hist.at[pl.ds(sid * rpt, rpt)],
                  deg_out.at[cid, pl.ds(sid * rpt, rpt)])


def _agg_kernel_body(cpw, rpt, hd, y0_hbm, y1_hbm, rc_hbm, z_hbm,
                     out0_hbm, out1_hbm, rcv, buf, acc, sems):
  cid = lax.axis_index("c")
  sid = lax.axis_index("s")

  pltpu.sync_copy(z_hbm, acc.at[pl.ds(sid * rpt, rpt)])
  pltpu.sync_copy(rc_hbm.at[sid], rcv)
  plsc.subcore_barrier()

  def prime(j, carry):
    @pl.when(cid == 0)
    def _():
      pltpu.async_copy(y0_hbm.at[rcv.at[0, j]], buf.at[j], sems.at[j])

    @pl.when(cid == 1)
    def _():
      pltpu.async_copy(y1_hbm.at[rcv.at[0, j]], buf.at[j], sems.at[j])

    return carry

  lax.fori_loop(0, 2, prime, 0)

  def body(j, carry):
    par = lax.rem(j, 2)
    pltpu.make_async_copy(y0_hbm.at[pl.ds(0, CHUNK)], buf.at[par],
                          sems.at[par]).wait()
    pltpu.sync_copy(buf.at[par], acc.at[rcv.at[1, j]], add=True)

    @pl.when(jnp.logical_and(j + 2 < cpw, cid == 0))
    def _():
      pltpu.async_copy(y0_hbm.at[rcv.at[0, j + 2]], buf.at[par],
                       sems.at[par])

    @pl.when(jnp.logical_and(j + 2 < cpw, cid == 1))
    def _():
      pltpu.async_copy(y1_hbm.at[rcv.at[0, j + 2]], buf.at[par],
                       sems.at[par])

    return carry

  lax.fori_loop(0, cpw, body, 0)
  plsc.subcore_barrier()

  @pl.when(cid == 0)
  def _():
    pltpu.sync_copy(acc.at[pl.ds(sid * rpt, rpt)],
                    out0_hbm.at[pl.ds(sid * rpt, rpt)])

  @pl.when(cid == 1)
  def _():
    pltpu.sync_copy(acc.at[pl.ds(sid * rpt, rpt)],
                    out1_hbm.at[pl.ds(sid * rpt, rpt)])


def _matmul_body(xb, degb, wb, yb0, yb1):
  deg = degb[0, :] + degb[1, :] + 1.0
  dis = lax.rsqrt(deg)
  res = jnp.dot(xb[...], wb[...],
                preferred_element_type=jnp.float32) * dis[:, None]
  hd = res.shape[1] // 2
  yb0[...] = res[:, :hd]
  yb1[...] = res[:, hd:]


def _finish_body(a0b, a1b, y0b, y1b, degb, bb, ab, zb):
  deg = degb[0, :] + degb[1, :] + 1.0
  dis = lax.rsqrt(deg)
  acc = jnp.concatenate([a0b[...] + y0b[...], a1b[...] + y1b[...]], axis=1)
  s = acc * dis[:, None] + bb[...]
  zb[...] = jnp.where(s >= 0, s, ab[...] * s)


def kernel(x, edge_index, W, b, alpha):
  n = x.shape[0]
  e = edge_index.shape[1]
  d = x.shape[1]
  hd = d // 2

  nacc = 10240
  rpt = nacc // NS

  cpw_deg = -(-e // (NW * CHUNK))
  epad_deg = NW * cpw_deg * CHUNK - e
  col_deg = jnp.concatenate(
      [edge_index[1],
       jnp.full((epad_deg,), n, jnp.int32)]).reshape(NW, cpw_deg, CHUNK)

  cpw = -(-e // (NS * CHUNK))
  cpw += cpw % 2
  epad = NS * cpw * CHUNK - e
  row = jnp.concatenate(
      [edge_index[0], jnp.zeros((epad,), jnp.int32)]).reshape(NS, 1, cpw,
                                                             CHUNK)
  col = jnp.concatenate(
      [edge_index[1], jnp.full((epad,), n, jnp.int32)]).reshape(NS, 1, cpw,
                                                                CHUNK)
  rc = jnp.concatenate([row, col], axis=1)

  x_pad = jnp.pad(x, ((0, nacc - n), (0, 0)))
  zeros_rows = jnp.zeros((rpt, hd), jnp.float32)

  mesh = plsc.VectorSubcoreMesh(**_MESH)

  deg_fn = pl.kernel(
      functools.partial(_deg_kernel_body, cpw_deg, rpt),
      out_type=jax.ShapeDtypeStruct((NC, nacc), jnp.float32),
      mesh=mesh,
      scratch_types=[
          pltpu.VMEM((cpw_deg, CHUNK), jnp.int32),
          pltpu.VMEM((CHUNK,), jnp.float32),
          pltpu.VMEM((rpt,), jnp.float32),
          pltpu.VMEM_SHARED((nacc,), jnp.float32),
      ])
  deg = deg_fn(col_deg)

  y0, y1 = pl.pallas_call(
      _matmul_body,
      grid=(nacc // 1024,),
      in_specs=[
          pl.BlockSpec((1024, d), lambda i: (i, 0)),
          pl.BlockSpec((NC, 1024), lambda i: (0, i)),
          pl.BlockSpec((d, d), lambda i: (0, 0)),
      ],
      out_specs=[
          pl.BlockSpec((1024, hd), lambda i: (i, 0)),
          pl.BlockSpec((1024, hd), lambda i: (i, 0)),
      ],
      out_shape=[
          jax.ShapeDtypeStruct((nacc, hd), jnp.float32),
          jax.ShapeDtypeStruct((nacc, hd), jnp.float32),
      ],
  )(x_pad, deg, W)

  agg_fn = pl.kernel(
      functools.partial(_agg_kernel_body, cpw, rpt, hd),
      out_type=[
          jax.ShapeDtypeStruct((nacc, hd), jnp.float32),
          jax.ShapeDtypeStruct((nacc, hd), jnp.float32),
      ],
      mesh=mesh,
      scratch_types=[
          pltpu.VMEM((2, cpw, CHUNK), jnp.int32),
          pltpu.VMEM((2, CHUNK, hd), jnp.float32),
          pltpu.VMEM_SHARED((nacc, hd), jnp.float32),
          pltpu.SemaphoreType.DMA((2,)),
      ],
      compiler_params=pltpu.CompilerParams(use_tc_tiling_on_sc=False))
  acc0, acc1 = agg_fn(y0, y1, rc, zeros_rows)

  z = pl.pallas_call(
      _finish_body,
      grid=(nacc // 1024,),
      in_specs=[
          pl.BlockSpec((1024, hd), lambda i: (i, 0)),
          pl.BlockSpec((1024, hd), lambda i: (i, 0)),
          pl.BlockSpec((1024, hd), lambda i: (i, 0)),
          pl.BlockSpec((1024, hd), lambda i: (i, 0)),
          pl.BlockSpec((NC, 1024), lambda i: (0, i)),
          pl.BlockSpec((1, d), lambda i: (0, 0)),
          pl.BlockSpec((1, d), lambda i: (0, 0)),
      ],
      out_specs=pl.BlockSpec((1024, d), lambda i: (i, 0)),
      out_shape=jax.ShapeDtypeStruct((nacc, d), jnp.float32),
  )(acc0, acc1, y0, y1, deg, b.reshape(1, d), alpha.reshape(1, d))

  return z[:n]
